# baseline (speedup 1.0000x reference)
import functools

import jax
import jax.numpy as jnp
from jax import lax
from jax.experimental import pallas as pl
from jax.experimental.pallas import tpu as pltpu
from jax.experimental.pallas import tpu_sc as plsc

N = 10000
E = 320000
D = 128

NW = 32
K = 80
EA = 163840
EB_N = 156160
NPAD = 10240
RPT = NPAD // 16

_mesh = plsc.VectorSubcoreMesh(core_axis_name="c", subcore_axis_name="s")


def _make_sc_gather(nchunk):
    e_half = NW * nchunk * K

    @functools.partial(
        pl.kernel,
        out_type=jax.ShapeDtypeStruct((e_half, D), jnp.float32),
        mesh=_mesh,
        scratch_types=[
            pltpu.VMEM((nchunk, K), jnp.int32),
            pltpu.VMEM((nchunk, K), jnp.int32),
            pltpu.VMEM((K, D), jnp.float32),
            pltpu.VMEM((K, D), jnp.float32),
            pltpu.VMEM((K, D), jnp.float32),
            pltpu.VMEM((K, D), jnp.float32),
            pltpu.VMEM((K, D), jnp.float32),
            pltpu.VMEM((K, D), jnp.float32),
            pltpu.SemaphoreType.DMA,
            pltpu.SemaphoreType.DMA,
            pltpu.SemaphoreType.DMA,
            pltpu.SemaphoreType.DMA,
        ],
    )
    def _sc_gather(p_hbm, q_hbm, didx_hbm, sidx_hbm, g_hbm,
                   di_all, si_all, pr_a, qr_a, pr_b, qr_b, oa, ob,
                   sem_a, sem_b, sem_oa, sem_ob):
        c = lax.axis_index("c")
        s = lax.axis_index("s")
        w = s * 2 + c

        pltpu.sync_copy(didx_hbm.at[w], di_all)
        pltpu.sync_copy(sidx_hbm.at[w], si_all)

        def start(j, pr, qr, sem):
            pltpu.async_copy(p_hbm.at[di_all.at[j]], pr, sem)
            pltpu.async_copy(q_hbm.at[si_all.at[j]], qr, sem)

        def drain(pr, qr, sem):
            pltpu.make_async_copy(g_hbm.at[pl.ds(0, K)], pr, sem).wait()
            pltpu.make_async_copy(g_hbm.at[pl.ds(0, K)], qr, sem).wait()

        def wait_out(o, sem_o):
            pltpu.make_async_copy(g_hbm.at[pl.ds(0, K)], o, sem_o).wait()

        def add(pr, qr, o):
            def row(t, carry):
                r = 2 * t
                for u in range(2):
                    for v in range(D // 16):
                        sl = pl.ds(v * 16, 16)
                        o[r + u, sl] = pr[r + u, sl] + qr[r + u, sl]
                return carry

            lax.fori_loop(0, K // 2, row, 0)

        def out(j, o, sem_o):
            base = (w * nchunk + j) * K
            pltpu.async_copy(o, g_hbm.at[pl.ds(base, K)], sem_o)

        start(0, pr_a, qr_a, sem_a)

        def body(t, carry):
            j0 = 2 * t
            start(j0 + 1, pr_b, qr_b, sem_b)
            drain(pr_a, qr_a, sem_a)

            @pl.when(t > 0)
            def _():
                wait_out(oa, sem_oa)

            add(pr_a, qr_a, oa)
            start(j0 + 2, pr_a, qr_a, sem_a)
            out(j0, oa, sem_oa)
            drain(pr_b, qr_b, sem_b)

            @pl.when(t > 0)
            def _():
                wait_out(ob, sem_ob)

            add(pr_b, qr_b, ob)
            out(j0 + 1, ob, sem_ob)
            return carry

        if nchunk % 2:
            lax.fori_loop(0, nchunk // 2, body, 0)
            drain(pr_a, qr_a, sem_a)
            wait_out(oa, sem_oa)
            add(pr_a, qr_a, oa)
            out(nchunk - 1, oa, sem_oa)
        else:
            lax.fori_loop(0, nchunk // 2 - 1, body, 0)
            j0 = nchunk - 2
            start(j0 + 1, pr_b, qr_b, sem_b)
            drain(pr_a, qr_a, sem_a)
            wait_out(oa, sem_oa)
            add(pr_a, qr_a, oa)
            out(j0, oa, sem_oa)
            drain(pr_b, qr_b, sem_b)
            wait_out(ob, sem_ob)
            add(pr_b, qr_b, ob)
            out(j0 + 1, ob, sem_ob)
        wait_out(oa, sem_oa)
        wait_out(ob, sem_ob)

    return _sc_gather


def _make_sc_scatter(nchunk):
    e_half = NW * nchunk * K

    @functools.partial(
        pl.kernel,
        out_type=jax.ShapeDtypeStruct((2, NPAD, D), jnp.float32),
        mesh=_mesh,
        scratch_types=[
            pltpu.VMEM((nchunk, K), jnp.int32),
            pltpu.VMEM((K, D), jnp.float32),
            pltpu.VMEM((K, D), jnp.float32),
            pltpu.VMEM_SHARED((NPAD, D), jnp.float32),
            pltpu.SemaphoreType.DMA,
            pltpu.SemaphoreType.DMA,
            pltpu.SemaphoreType.DMA,
            pltpu.SemaphoreType.DMA,
        ],
    )
    def _sc_scatter(ue_hbm, sidx_hbm, init_hbm, out_hbm, si_all, row_a, row_b,
                    acc, sem_a, sem_b, sem_sa, sem_sb):
        c = lax.axis_index("c")
        s = lax.axis_index("s")
        w = s * 2 + c
        r0 = s * RPT
        pltpu.sync_copy(init_hbm.at[c, pl.ds(r0, RPT)], acc.at[pl.ds(r0, RPT)])
        pltpu.sync_copy(sidx_hbm.at[w], si_all)
        plsc.subcore_barrier()

        def fetch(j, row, sem):
            base = (w * nchunk + j) * K
            pltpu.async_copy(ue_hbm.at[pl.ds(base, K)], row, sem)

        def wait_fetch(row, sem):
            pltpu.make_async_copy(ue_hbm.at[pl.ds(0, K)], row, sem).wait()

        def scat(j, row, sem_s):
            pltpu.async_copy(row, acc.at[si_all.at[j]], sem_s, add=True)

        def wait_scat(row, sem_s):
            pltpu.make_async_copy(ue_hbm.at[pl.ds(0, K)], row, sem_s).wait()

        fetch(0, row_a, sem_a)
        fetch(1, row_b, sem_b)

        def body(t, carry):
            j0 = 2 * t
            wait_fetch(row_a, sem_a)
            scat(j0, row_a, sem_sa)
            wait_fetch(row_b, sem_b)
            scat(j0 + 1, row_b, sem_sb)
            wait_scat(row_a, sem_sa)
            fetch(j0 + 2, row_a, sem_a)
            wait_scat(row_b, sem_sb)
            fetch(j0 + 3, row_b, sem_b)
            return carry

        if nchunk % 2:
            lax.fori_loop(0, (nchunk - 1) // 2 - 1, body, 0)
            j0 = nchunk - 3
            wait_fetch(row_a, sem_a)
            scat(j0, row_a, sem_sa)
            wait_fetch(row_b, sem_b)
            scat(j0 + 1, row_b, sem_sb)
            wait_scat(row_a, sem_sa)
            fetch(j0 + 2, row_a, sem_a)
            wait_fetch(row_a, sem_a)
            scat(nchunk - 1, row_a, sem_sa)
            wait_scat(row_a, sem_sa)
            wait_scat(row_b, sem_sb)
        else:
            lax.fori_loop(0, nchunk // 2 - 1, body, 0)
            j0 = nchunk - 2
            wait_fetch(row_a, sem_a)
            scat(j0, row_a, sem_sa)
            wait_fetch(row_b, sem_b)
            scat(j0 + 1, row_b, sem_sb)
            wait_scat(row_a, sem_sa)
            wait_scat(row_b, sem_sb)
        plsc.subcore_barrier()
        pltpu.sync_copy(acc.at[pl.ds(r0, RPT)], out_hbm.at[c, pl.ds(r0, RPT)])

    return _sc_scatter


_sc_gather_a = _make_sc_gather(EA // (NW * K))
_sc_gather_b = _make_sc_gather(EB_N // (NW * K))
_sc_scatter_a = _make_sc_scatter(EA // (NW * K))
_sc_scatter_b = _make_sc_scatter(EB_N // (NW * K))


def _ln(h, g_ref, b_ref):
    m = jnp.mean(h, axis=-1, keepdims=True)
    d = h - m
    var = jnp.mean(d * d, axis=-1, keepdims=True)
    return d * lax.rsqrt(var + 1e-5) * g_ref[...] + b_ref[...]


def _edge_body(g_ref, ea_ref, wet_ref, eb0_ref, ew1t_ref, eb1_ref,
               eg_ref, ebe_ref, ue_ref):
    ea = ea_ref[...]
    h = (g_ref[...]
         + jnp.dot(ea, wet_ref[...], preferred_element_type=jnp.float32)
         + eb0_ref[...])
    h = jnp.maximum(h, 0.0)
    h = jnp.dot(h, ew1t_ref[...], preferred_element_type=jnp.float32) + eb1_ref[...]
    ue_ref[...] = ea + _ln(h, eg_ref, ebe_ref)


EBLK = 2560

_full = lambda i: (0, 0)
_blk = lambda i: (i, 0)


def _make_edge_call(e_half, ea_blk_off):
    return pl.pallas_call(
        _edge_body,
        grid=(e_half // EBLK,),
        in_specs=[
            pl.BlockSpec((EBLK, D), _blk),
            pl.BlockSpec((EBLK, D), lambda i: (i + ea_blk_off, 0)),
            pl.BlockSpec((D, D), _full),
            pl.BlockSpec((1, D), _full),
            pl.BlockSpec((D, D), _full),
            pl.BlockSpec((1, D), _full),
            pl.BlockSpec((1, D), _full),
            pl.BlockSpec((1, D), _full),
        ],
        out_specs=pl.BlockSpec((EBLK, D), _blk),
        out_shape=jax.ShapeDtypeStruct((e_half, D), jnp.float32),
    )


_edge_call_a0 = _make_edge_call(EA, 0)
_edge_call_b0 = _make_edge_call(EB_N, EA // EBLK)
_edge_call_a = _make_edge_call(EA, 0)
_edge_call_b = _make_edge_call(EB_N, 0)


def _node_body(x_ref, o0_ref, o1_ref, at_ref, bt_ref, nb0_ref, n1t_ref,
               nb1_ref, ng_ref, nbe_ref, wit_ref, wjt_ref,
               xo_ref, p_ref, q_ref):
    x = x_ref[...]
    o = o0_ref[...] + o1_ref[...]
    g = (jnp.dot(x, at_ref[...], preferred_element_type=jnp.float32)
         + jnp.dot(o, bt_ref[...], preferred_element_type=jnp.float32)
         + nb0_ref[...])
    g = jnp.maximum(g, 0.0)
    g = jnp.dot(g, n1t_ref[...], preferred_element_type=jnp.float32) + nb1_ref[...]
    xn = x + _ln(g, ng_ref, nbe_ref)
    xo_ref[...] = xn
    p_ref[...] = jnp.dot(xn, wit_ref[...], preferred_element_type=jnp.float32)
    q_ref[...] = jnp.dot(xn, wjt_ref[...], preferred_element_type=jnp.float32)


NB = 2000

_node_call = pl.pallas_call(
    _node_body,
    grid=(N // NB,),
    in_specs=[
        pl.BlockSpec((NB, D), _blk),
        pl.BlockSpec((NB, D), _blk),
        pl.BlockSpec((NB, D), _blk),
        pl.BlockSpec((D, D), _full),
        pl.BlockSpec((D, D), _full),
        pl.BlockSpec((1, D), _full),
        pl.BlockSpec((D, D), _full),
        pl.BlockSpec((1, D), _full),
        pl.BlockSpec((1, D), _full),
        pl.BlockSpec((1, D), _full),
        pl.BlockSpec((D, D), _full),
        pl.BlockSpec((D, D), _full),
    ],
    out_specs=[
        pl.BlockSpec((NB, D), _blk),
        pl.BlockSpec((NB, D), _blk),
        pl.BlockSpec((NB, D), _blk),
    ],
    out_shape=[
        jax.ShapeDtypeStruct((N, D), jnp.float32),
        jax.ShapeDtypeStruct((N, D), jnp.float32),
        jax.ShapeDtypeStruct((N, D), jnp.float32),
    ],
)


def _pq_body(x_ref, wit_ref, wjt_ref, p_ref, q_ref):
    x = x_ref[...]
    p_ref[...] = jnp.dot(x, wit_ref[...], preferred_element_type=jnp.float32)
    q_ref[...] = jnp.dot(x, wjt_ref[...], preferred_element_type=jnp.float32)


_pq_call = pl.pallas_call(
    _pq_body,
    grid=(N // NB,),
    in_specs=[
        pl.BlockSpec((NB, D), _blk),
        pl.BlockSpec((D, D), _full),
        pl.BlockSpec((D, D), _full),
    ],
    out_specs=[
        pl.BlockSpec((NB, D), _blk),
        pl.BlockSpec((NB, D), _blk),
    ],
    out_shape=[
        jax.ShapeDtypeStruct((N, D), jnp.float32),
        jax.ShapeDtypeStruct((N, D), jnp.float32),
    ],
)


def kernel(x, edge_index, edge_attr, params):
    src = edge_index[0].astype(jnp.int32)
    dst = edge_index[1].astype(jnp.int32)
    nca = EA // (NW * K)
    ncb = EB_N // (NW * K)
    src_a = src[:EA].reshape(NW, nca, K)
    dst_a = dst[:EA].reshape(NW, nca, K)
    src_b = src[EA:].reshape(NW, ncb, K)
    dst_b = dst[EA:].reshape(NW, ncb, K)
    zeros = jnp.zeros((2, NPAD, D), jnp.float32)

    p0 = params[0]
    P, Q = _pq_call(x, p0['ew0'][:, :D].T, p0['ew0'][:, D:2 * D].T)

    ea_a = edge_attr
    ea_b = edge_attr
    for li, p in enumerate(params):
        wet, eb0 = p['ew0'][:, 2 * D:].T, p['eb0'][None]
        ew1t, eb1 = p['ew1'].T, p['eb1'][None]
        eg, ebe = p['eg'][None], p['ebeta'][None]
        ec_a = _edge_call_a0 if li == 0 else _edge_call_a
        ec_b = _edge_call_b0 if li == 0 else _edge_call_b
        g_a = _sc_gather_a(P, Q, dst_a, src_a)
        g_b = _sc_gather_b(P, Q, dst_b, src_b)
        ue_a = ec_a(g_a, ea_a, wet, eb0, ew1t, eb1, eg, ebe)
        ue_b = ec_b(g_b, ea_b, wet, eb0, ew1t, eb1, eg, ebe)
        part = _sc_scatter_a(ue_a, src_a, zeros)
        part = _sc_scatter_b(ue_b, src_b, part)
        if li + 1 < len(params):
            nxt = params[li + 1]
            wit, wjt = nxt['ew0'][:, :D].T, nxt['ew0'][:, D:2 * D].T
        else:
            wit = wjt = jnp.zeros((D, D), jnp.float32)
        x, P, Q = _node_call(x, part[0, :N], part[1, :N],
                             p['nw0'][:, :D].T, p['nw0'][:, D:].T,
                             p['nb0'][None], p['nw1'].T, p['nb1'][None],
                             p['ng'][None], p['nbeta'][None], wit, wjt)
        ea_a, ea_b = ue_a, ue_b
    return x, jnp.concatenate([ea_a, ea_b], axis=0)


# speedup vs baseline: 1.0154x; 1.0154x over previous
import functools

import jax
import jax.numpy as jnp
from jax import lax
from jax.experimental import pallas as pl
from jax.experimental.pallas import tpu as pltpu
from jax.experimental.pallas import tpu_sc as plsc

N = 10000
E = 320000
D = 128

NW = 32
K = 80
EA = 163840
EB_N = 156160
NPAD = 10240
RPT = NPAD // 16

_mesh = plsc.VectorSubcoreMesh(core_axis_name="c", subcore_axis_name="s")


def _make_sc_gather(nchunk):
    e_half = NW * nchunk * K

    @functools.partial(
        pl.kernel,
        out_type=jax.ShapeDtypeStruct((e_half, D), jnp.float32),
        mesh=_mesh,
        scratch_types=[
            pltpu.VMEM((nchunk, K), jnp.int32),
            pltpu.VMEM((nchunk, K), jnp.int32),
            pltpu.VMEM((K, D), jnp.float32),
            pltpu.VMEM((K, D), jnp.float32),
            pltpu.VMEM((K, D), jnp.float32),
            pltpu.VMEM((K, D), jnp.float32),
            pltpu.VMEM((K, D), jnp.float32),
            pltpu.VMEM((K, D), jnp.float32),
            pltpu.SemaphoreType.DMA,
            pltpu.SemaphoreType.DMA,
            pltpu.SemaphoreType.DMA,
            pltpu.SemaphoreType.DMA,
        ],
    )
    def _sc_gather(p_hbm, q_hbm, didx_hbm, sidx_hbm, g_hbm,
                   di_all, si_all, pr_a, qr_a, pr_b, qr_b, oa, ob,
                   sem_a, sem_b, sem_oa, sem_ob):
        c = lax.axis_index("c")
        s = lax.axis_index("s")
        w = s * 2 + c

        pltpu.sync_copy(didx_hbm.at[w], di_all)
        pltpu.sync_copy(sidx_hbm.at[w], si_all)

        def start(j, pr, qr, sem):
            pltpu.async_copy(p_hbm.at[di_all.at[j]], pr, sem)
            pltpu.async_copy(q_hbm.at[si_all.at[j]], qr, sem)

        def drain(pr, qr, sem):
            pltpu.make_async_copy(g_hbm.at[pl.ds(0, K)], pr, sem).wait()
            pltpu.make_async_copy(g_hbm.at[pl.ds(0, K)], qr, sem).wait()

        def wait_out(o, sem_o):
            pltpu.make_async_copy(g_hbm.at[pl.ds(0, K)], o, sem_o).wait()

        def add(pr, qr, o):
            def row(t, carry):
                r = 2 * t
                for u in range(2):
                    for v in range(D // 16):
                        sl = pl.ds(v * 16, 16)
                        o[r + u, sl] = pr[r + u, sl] + qr[r + u, sl]
                return carry

            lax.fori_loop(0, K // 2, row, 0)

        def out(j, o, sem_o):
            base = (w * nchunk + j) * K
            pltpu.async_copy(o, g_hbm.at[pl.ds(base, K)], sem_o)

        start(0, pr_a, qr_a, sem_a)

        def body(t, carry):
            j0 = 2 * t
            start(j0 + 1, pr_b, qr_b, sem_b)
            drain(pr_a, qr_a, sem_a)

            @pl.when(t > 0)
            def _():
                wait_out(oa, sem_oa)

            add(pr_a, qr_a, oa)
            start(j0 + 2, pr_a, qr_a, sem_a)
            out(j0, oa, sem_oa)
            drain(pr_b, qr_b, sem_b)

            @pl.when(t > 0)
            def _():
                wait_out(ob, sem_ob)

            add(pr_b, qr_b, ob)
            out(j0 + 1, ob, sem_ob)
            return carry

        if nchunk % 2:
            lax.fori_loop(0, nchunk // 2, body, 0)
            drain(pr_a, qr_a, sem_a)
            wait_out(oa, sem_oa)
            add(pr_a, qr_a, oa)
            out(nchunk - 1, oa, sem_oa)
        else:
            lax.fori_loop(0, nchunk // 2 - 1, body, 0)
            j0 = nchunk - 2
            start(j0 + 1, pr_b, qr_b, sem_b)
            drain(pr_a, qr_a, sem_a)
            wait_out(oa, sem_oa)
            add(pr_a, qr_a, oa)
            out(j0, oa, sem_oa)
            drain(pr_b, qr_b, sem_b)
            wait_out(ob, sem_ob)
            add(pr_b, qr_b, ob)
            out(j0 + 1, ob, sem_ob)
        wait_out(oa, sem_oa)
        wait_out(ob, sem_ob)

    return _sc_gather


def _make_sc_scatter(nchunk):
    e_half = NW * nchunk * K

    @functools.partial(
        pl.kernel,
        out_type=jax.ShapeDtypeStruct((2, NPAD, D), jnp.float32),
        mesh=_mesh,
        scratch_types=[
            pltpu.VMEM((nchunk, K), jnp.int32),
            pltpu.VMEM((K, D), jnp.float32),
            pltpu.VMEM((K, D), jnp.float32),
            pltpu.VMEM_SHARED((NPAD, D), jnp.float32),
            pltpu.SemaphoreType.DMA,
            pltpu.SemaphoreType.DMA,
        ],
    )
    def _sc_scatter(ue_hbm, sidx_hbm, init_hbm, out_hbm, si_all, row_a, row_b,
                    acc, sem_a, sem_b):
        c = lax.axis_index("c")
        s = lax.axis_index("s")
        w = s * 2 + c
        r0 = s * RPT
        pltpu.sync_copy(init_hbm.at[c, pl.ds(r0, RPT)], acc.at[pl.ds(r0, RPT)])
        pltpu.sync_copy(sidx_hbm.at[w], si_all)
        plsc.subcore_barrier()

        def fetch(j, row, sem):
            base = (w * nchunk + j) * K
            pltpu.async_copy(ue_hbm.at[pl.ds(base, K)], row, sem)

        def scat(j, row, sem):
            pltpu.make_async_copy(ue_hbm.at[pl.ds(0, K)], row, sem).wait()
            pltpu.sync_copy(row, acc.at[si_all.at[j]], add=True)

        fetch(0, row_a, sem_a)

        def body(t, carry):
            j0 = 2 * t
            fetch(j0 + 1, row_b, sem_b)
            scat(j0, row_a, sem_a)
            fetch(j0 + 2, row_a, sem_a)
            scat(j0 + 1, row_b, sem_b)
            return carry

        if nchunk % 2:
            lax.fori_loop(0, nchunk // 2, body, 0)
            scat(nchunk - 1, row_a, sem_a)
        else:
            lax.fori_loop(0, nchunk // 2 - 1, body, 0)
            j0 = nchunk - 2
            fetch(j0 + 1, row_b, sem_b)
            scat(j0, row_a, sem_a)
            scat(j0 + 1, row_b, sem_b)
        plsc.subcore_barrier()
        pltpu.sync_copy(acc.at[pl.ds(r0, RPT)], out_hbm.at[c, pl.ds(r0, RPT)])

    return _sc_scatter


_sc_gather_a = _make_sc_gather(EA // (NW * K))
_sc_gather_b = _make_sc_gather(EB_N // (NW * K))
_sc_scatter_a = _make_sc_scatter(EA // (NW * K))
_sc_scatter_b = _make_sc_scatter(EB_N // (NW * K))


def _ln(h, g_ref, b_ref):
    m = jnp.mean(h, axis=-1, keepdims=True)
    d = h - m
    var = jnp.mean(d * d, axis=-1, keepdims=True)
    return d * lax.rsqrt(var + 1e-5) * g_ref[...] + b_ref[...]


def _edge_body(g_ref, ea_ref, wet_ref, eb0_ref, ew1t_ref, eb1_ref,
               eg_ref, ebe_ref, ue_ref):
    ea = ea_ref[...]
    h = (g_ref[...]
         + jnp.dot(ea, wet_ref[...], preferred_element_type=jnp.float32)
         + eb0_ref[...])
    h = jnp.maximum(h, 0.0)
    h = jnp.dot(h, ew1t_ref[...], preferred_element_type=jnp.float32) + eb1_ref[...]
    ue_ref[...] = ea + _ln(h, eg_ref, ebe_ref)


EBLK = 2560

_full = lambda i: (0, 0)
_blk = lambda i: (i, 0)


def _make_edge_call(e_half, ea_blk_off):
    return pl.pallas_call(
        _edge_body,
        grid=(e_half // EBLK,),
        in_specs=[
            pl.BlockSpec((EBLK, D), _blk),
            pl.BlockSpec((EBLK, D), lambda i: (i + ea_blk_off, 0)),
            pl.BlockSpec((D, D), _full),
            pl.BlockSpec((1, D), _full),
            pl.BlockSpec((D, D), _full),
            pl.BlockSpec((1, D), _full),
            pl.BlockSpec((1, D), _full),
            pl.BlockSpec((1, D), _full),
        ],
        out_specs=pl.BlockSpec((EBLK, D), _blk),
        out_shape=jax.ShapeDtypeStruct((e_half, D), jnp.float32),
    )


_edge_call_a0 = _make_edge_call(EA, 0)
_edge_call_b0 = _make_edge_call(EB_N, EA // EBLK)
_edge_call_a = _make_edge_call(EA, 0)
_edge_call_b = _make_edge_call(EB_N, 0)


def _node_body(x_ref, o0_ref, o1_ref, at_ref, bt_ref, nb0_ref, n1t_ref,
               nb1_ref, ng_ref, nbe_ref, wit_ref, wjt_ref,
               xo_ref, p_ref, q_ref):
    x = x_ref[...]
    o = o0_ref[...] + o1_ref[...]
    g = (jnp.dot(x, at_ref[...], preferred_element_type=jnp.float32)
         + jnp.dot(o, bt_ref[...], preferred_element_type=jnp.float32)
         + nb0_ref[...])
    g = jnp.maximum(g, 0.0)
    g = jnp.dot(g, n1t_ref[...], preferred_element_type=jnp.float32) + nb1_ref[...]
    xn = x + _ln(g, ng_ref, nbe_ref)
    xo_ref[...] = xn
    p_ref[...] = jnp.dot(xn, wit_ref[...], preferred_element_type=jnp.float32)
    q_ref[...] = jnp.dot(xn, wjt_ref[...], preferred_element_type=jnp.float32)


NB = 2000

_node_call = pl.pallas_call(
    _node_body,
    grid=(N // NB,),
    in_specs=[
        pl.BlockSpec((NB, D), _blk),
        pl.BlockSpec((NB, D), _blk),
        pl.BlockSpec((NB, D), _blk),
        pl.BlockSpec((D, D), _full),
        pl.BlockSpec((D, D), _full),
        pl.BlockSpec((1, D), _full),
        pl.BlockSpec((D, D), _full),
        pl.BlockSpec((1, D), _full),
        pl.BlockSpec((1, D), _full),
        pl.BlockSpec((1, D), _full),
        pl.BlockSpec((D, D), _full),
        pl.BlockSpec((D, D), _full),
    ],
    out_specs=[
        pl.BlockSpec((NB, D), _blk),
        pl.BlockSpec((NB, D), _blk),
        pl.BlockSpec((NB, D), _blk),
    ],
    out_shape=[
        jax.ShapeDtypeStruct((N, D), jnp.float32),
        jax.ShapeDtypeStruct((N, D), jnp.float32),
        jax.ShapeDtypeStruct((N, D), jnp.float32),
    ],
)


def _pq_body(x_ref, wit_ref, wjt_ref, p_ref, q_ref):
    x = x_ref[...]
    p_ref[...] = jnp.dot(x, wit_ref[...], preferred_element_type=jnp.float32)
    q_ref[...] = jnp.dot(x, wjt_ref[...], preferred_element_type=jnp.float32)


_pq_call = pl.pallas_call(
    _pq_body,
    grid=(N // NB,),
    in_specs=[
        pl.BlockSpec((NB, D), _blk),
        pl.BlockSpec((D, D), _full),
        pl.BlockSpec((D, D), _full),
    ],
    out_specs=[
        pl.BlockSpec((NB, D), _blk),
        pl.BlockSpec((NB, D), _blk),
    ],
    out_shape=[
        jax.ShapeDtypeStruct((N, D), jnp.float32),
        jax.ShapeDtypeStruct((N, D), jnp.float32),
    ],
)


def kernel(x, edge_index, edge_attr, params):
    src = edge_index[0].astype(jnp.int32)
    dst = edge_index[1].astype(jnp.int32)
    nca = EA // (NW * K)
    ncb = EB_N // (NW * K)
    src_a = src[:EA].reshape(NW, nca, K)
    dst_a = dst[:EA].reshape(NW, nca, K)
    src_b = src[EA:].reshape(NW, ncb, K)
    dst_b = dst[EA:].reshape(NW, ncb, K)
    zeros = jnp.zeros((2, NPAD, D), jnp.float32)

    p0 = params[0]
    P, Q = _pq_call(x, p0['ew0'][:, :D].T, p0['ew0'][:, D:2 * D].T)

    ea_a = edge_attr
    ea_b = edge_attr
    for li, p in enumerate(params):
        wet, eb0 = p['ew0'][:, 2 * D:].T, p['eb0'][None]
        ew1t, eb1 = p['ew1'].T, p['eb1'][None]
        eg, ebe = p['eg'][None], p['ebeta'][None]
        ec_a = _edge_call_a0 if li == 0 else _edge_call_a
        ec_b = _edge_call_b0 if li == 0 else _edge_call_b
        g_a = _sc_gather_a(P, Q, dst_a, src_a)
        g_b = _sc_gather_b(P, Q, dst_b, src_b)
        ue_a = ec_a(g_a, ea_a, wet, eb0, ew1t, eb1, eg, ebe)
        ue_b = ec_b(g_b, ea_b, wet, eb0, ew1t, eb1, eg, ebe)
        part = _sc_scatter_a(ue_a, src_a, zeros)
        part = _sc_scatter_b(ue_b, src_b, part)
        if li + 1 < len(params):
            nxt = params[li + 1]
            wit, wjt = nxt['ew0'][:, :D].T, nxt['ew0'][:, D:2 * D].T
        else:
            wit = wjt = jnp.zeros((D, D), jnp.float32)
        x, P, Q = _node_call(x, part[0, :N], part[1, :N],
                             p['nw0'][:, :D].T, p['nw0'][:, D:].T,
                             p['nb0'][None], p['nw1'].T, p['nb1'][None],
                             p['ng'][None], p['nbeta'][None], wit, wjt)
        ea_a, ea_b = ue_a, ue_b
    return x, jnp.concatenate([ea_a, ea_b], axis=0)
